# baseline (speedup 1.0000x reference)
import jax
import jax.numpy as jnp
from jax.experimental import pallas as pl
from jax.experimental.pallas import tpu as pltpu

B, N, F, H, C = 128, 512, 256, 256, 10

BB = 8
HALF = B // 2


def _gcn_body(x1_ref, a1_ref, x2_ref, a2_ref,
              w1_ref, b1_ref, w2_ref, b2_ref, o1_ref, o2_ref):
    w1 = w1_ref[...].astype(jnp.bfloat16)
    w2 = w2_ref[...].astype(jnp.bfloat16)
    for x_ref, a_ref, o_ref in ((x1_ref, a1_ref, o1_ref),
                                (x2_ref, a2_ref, o2_ref)):
        for i in range(BB):
            a = a_ref[i].astype(jnp.bfloat16)
            h = jnp.dot(x_ref[i].astype(jnp.bfloat16), w1,
                        preferred_element_type=jnp.float32)
            h = h + b1_ref[...]
            h = jnp.dot(a, h.astype(jnp.bfloat16),
                        preferred_element_type=jnp.float32)
            h = jnp.maximum(h, 0.0)
            h = jnp.dot(h.astype(jnp.bfloat16), w2,
                        preferred_element_type=jnp.float32)
            h = h + b2_ref[...]
            o_ref[i] = jnp.dot(a, h.astype(jnp.bfloat16),
                               preferred_element_type=jnp.float32)


def kernel(x, adj, W1, b1, W2, b2):
    b1r = b1.reshape(1, H)
    b2r = b2.reshape(1, C)
    nsteps = HALF // BB
    lo = lambda b: (b, 0, 0)
    hi = lambda b: (nsteps + b, 0, 0)
    const2 = lambda b: (0, 0)
    outs = pl.pallas_call(
        _gcn_body,
        grid=(nsteps,),
        in_specs=[
            pl.BlockSpec((BB, N, F), lo),
            pl.BlockSpec((BB, N, N), lo),
            pl.BlockSpec((BB, N, F), hi),
            pl.BlockSpec((BB, N, N), hi),
            pl.BlockSpec((F, H), const2),
            pl.BlockSpec((1, H), const2),
            pl.BlockSpec((H, C), const2),
            pl.BlockSpec((1, C), const2),
        ],
        out_specs=[pl.BlockSpec((BB, N, C), lo), pl.BlockSpec((BB, N, C), lo)],
        out_shape=[jax.ShapeDtypeStruct((HALF, N, C), jnp.float32),
                   jax.ShapeDtypeStruct((HALF, N, C), jnp.float32)],
        compiler_params=pltpu.CompilerParams(dimension_semantics=("arbitrary",)),
    )(x, adj, x, adj, W1, b1r, W2, b2r)
    return jnp.concatenate(outs, axis=0)[None]
